# speedup vs baseline: 6.3229x; 6.3229x over previous
import functools

import jax
import jax.numpy as jnp
from jax import lax
from jax.experimental import pallas as pl
from jax.experimental.pallas import tpu as pltpu
from jax.experimental.pallas import tpu_sc as plsc

NC = 2
NS = 16
NW = NC * NS

CHUNK = 128


def _make_sc_gather(total, d):
    per_w = total // NW
    nchunks = per_w // CHUNK
    mesh = plsc.VectorSubcoreMesh(core_axis_name="c", subcore_axis_name="s")

    @functools.partial(
        pl.kernel,
        mesh=mesh,
        out_type=jax.ShapeDtypeStruct((total, d), jnp.float32),
        scratch_types=[
            pltpu.VMEM((nchunks, CHUNK), jnp.int32),
            pltpu.VMEM((CHUNK, d), jnp.float32),
            pltpu.SemaphoreType.DMA,
        ],
    )
    def gather_kernel(idx_hbm, table_hbm, out_hbm, idx_v, rows_v, sem):
        wid = lax.axis_index("s") * NC + lax.axis_index("c")
        base = wid * per_w
        pltpu.sync_copy(idx_hbm.at[wid], idx_v)

        def body(j, carry):
            pltpu.async_copy(table_hbm.at[idx_v.at[j]], rows_v, sem).wait()
            pltpu.sync_copy(rows_v, out_hbm.at[pl.ds(base + j * CHUNK, CHUNK)])
            return carry

        lax.fori_loop(0, nchunks, body, 0)

    return gather_kernel


def kernel(x, table):
    total = x.shape[0] * x.shape[1]
    d = table.shape[1]
    idx = x.astype(jnp.int32).reshape(NW, total // (NW * CHUNK), CHUNK)
    out = _make_sc_gather(total, d)(idx, table)
    return out.reshape(x.shape[0], x.shape[1], d)


# speedup vs baseline: 7.5402x; 1.1925x over previous
import functools

import jax
import jax.numpy as jnp
from jax import lax
from jax.experimental import pallas as pl
from jax.experimental.pallas import tpu as pltpu
from jax.experimental.pallas import tpu_sc as plsc

NC = 2
NS = 16
NW = NC * NS

CHUNK = 128


def _make_sc_gather(total, d):
    per_w = total // NW
    nchunks = per_w // CHUNK
    mesh = plsc.VectorSubcoreMesh(core_axis_name="c", subcore_axis_name="s")

    @functools.partial(
        pl.kernel,
        mesh=mesh,
        out_type=jax.ShapeDtypeStruct((total, d), jnp.float32),
        scratch_types=[
            pltpu.VMEM((nchunks, CHUNK), jnp.int32),
            pltpu.VMEM((2, CHUNK, d), jnp.float32),
            pltpu.SemaphoreType.DMA,
        ],
    )
    def gather_kernel(idx_hbm, table_hbm, out_hbm, idx_v, rows_v, gsem):
        wid = lax.axis_index("s") * NC + lax.axis_index("c")
        base = wid * per_w
        pltpu.sync_copy(idx_hbm.at[wid], idx_v)

        pltpu.async_copy(table_hbm.at[idx_v.at[0]], rows_v.at[0], gsem)

        def body(j, carry):
            b = lax.rem(j, 2)
            pltpu.make_async_copy(
                table_hbm.at[idx_v.at[j]], rows_v.at[b], gsem
            ).wait()

            @pl.when(j + 1 < nchunks)
            def _():
                pltpu.async_copy(
                    table_hbm.at[idx_v.at[j + 1]], rows_v.at[1 - b], gsem
                )

            pltpu.sync_copy(
                rows_v.at[b], out_hbm.at[pl.ds(base + j * CHUNK, CHUNK)]
            )
            return carry

        lax.fori_loop(0, nchunks, body, 0)

    return gather_kernel


def kernel(x, table):
    total = x.shape[0] * x.shape[1]
    d = table.shape[1]
    idx = x.astype(jnp.int32).reshape(NW, total // (NW * CHUNK), CHUNK)
    out = _make_sc_gather(total, d)(idx, table)
    return out.reshape(x.shape[0], x.shape[1], d)


# speedup vs baseline: 9.2172x; 1.2224x over previous
import functools

import jax
import jax.numpy as jnp
from jax import lax
from jax.experimental import pallas as pl
from jax.experimental.pallas import tpu as pltpu
from jax.experimental.pallas import tpu_sc as plsc

NC = 2
NS = 16
NW = NC * NS

CHUNK = 128
NBUF = 4
PRIME = 2


def _make_sc_gather(total, d):
    per_w = total // NW
    nchunks = per_w // CHUNK
    ngroups = nchunks // NBUF
    mesh = plsc.VectorSubcoreMesh(core_axis_name="c", subcore_axis_name="s")

    @functools.partial(
        pl.kernel,
        mesh=mesh,
        out_type=jax.ShapeDtypeStruct((total, d), jnp.float32),
        scratch_types=[
            pltpu.VMEM((nchunks, CHUNK), jnp.int32),
            pltpu.VMEM((NBUF, CHUNK, d), jnp.float32),
        ]
        + [pltpu.SemaphoreType.DMA] * (2 * NBUF),
    )
    def gather_kernel(idx_hbm, table_hbm, out_hbm, idx_v, rows_v, *sems):
        gsems = sems[:NBUF]
        wsems = sems[NBUF:]
        wid = lax.axis_index("s") * NC + lax.axis_index("c")
        base = wid * per_w
        pltpu.sync_copy(idx_hbm.at[wid], idx_v)

        for b in range(PRIME):
            pltpu.async_copy(table_hbm.at[idx_v.at[b]], rows_v.at[b], gsems[b])

        def group(jo, carry):
            for b in range(NBUF):
                j = jo * NBUF + b
                pltpu.make_async_copy(
                    table_hbm.at[idx_v.at[j]], rows_v.at[b], gsems[b]
                ).wait()
                pltpu.async_copy(
                    rows_v.at[b],
                    out_hbm.at[pl.ds(base + j * CHUNK, CHUNK)],
                    wsems[b],
                )
                jn = j + PRIME
                bn = (b + PRIME) % NBUF

                @pl.when(jn < nchunks)
                def _():
                    @pl.when(j >= NBUF - PRIME)
                    def _():
                        pltpu.make_async_copy(
                            rows_v.at[bn],
                            out_hbm.at[pl.ds(base, CHUNK)],
                            wsems[bn],
                        ).wait()

                    pltpu.async_copy(
                        table_hbm.at[idx_v.at[jn]], rows_v.at[bn], gsems[bn]
                    )

            return carry

        lax.fori_loop(0, ngroups, group, 0)

        for b in range(NBUF):
            pltpu.make_async_copy(
                rows_v.at[b], out_hbm.at[pl.ds(base, CHUNK)], wsems[b]
            ).wait()

    return gather_kernel


def kernel(x, table):
    total = x.shape[0] * x.shape[1]
    d = table.shape[1]
    idx = x.astype(jnp.int32).reshape(NW, total // (NW * CHUNK), CHUNK)
    out = _make_sc_gather(total, d)(idx, table)
    return out.reshape(x.shape[0], x.shape[1], d)


# speedup vs baseline: 9.2408x; 1.0026x over previous
import functools

import jax
import jax.numpy as jnp
from jax import lax
from jax.experimental import pallas as pl
from jax.experimental.pallas import tpu as pltpu
from jax.experimental.pallas import tpu_sc as plsc

NC = 2
NS = 16
NW = NC * NS

CHUNK = 128
NBUF = 5
PRIME = 3


def _make_sc_gather(total, d):
    per_w = total // NW
    nchunks = per_w // CHUNK
    ngroups = nchunks // NBUF
    mesh = plsc.VectorSubcoreMesh(core_axis_name="c", subcore_axis_name="s")

    @functools.partial(
        pl.kernel,
        mesh=mesh,
        out_type=jax.ShapeDtypeStruct((total, d), jnp.float32),
        scratch_types=[
            pltpu.VMEM((nchunks, CHUNK), jnp.int32),
            pltpu.VMEM((NBUF, CHUNK, d), jnp.float32),
        ]
        + [pltpu.SemaphoreType.DMA] * (2 * NBUF),
    )
    def gather_kernel(idx_hbm, table_hbm, out_hbm, idx_v, rows_v, *sems):
        gsems = sems[:NBUF]
        wsems = sems[NBUF:]
        wid = lax.axis_index("s") * NC + lax.axis_index("c")
        base = wid * per_w
        pltpu.sync_copy(idx_hbm.at[wid], idx_v)

        for b in range(PRIME):
            pltpu.async_copy(table_hbm.at[idx_v.at[b]], rows_v.at[b], gsems[b])

        def group(jo, carry):
            for b in range(NBUF):
                j = jo * NBUF + b
                pltpu.make_async_copy(
                    table_hbm.at[idx_v.at[j]], rows_v.at[b], gsems[b]
                ).wait()
                pltpu.async_copy(
                    rows_v.at[b],
                    out_hbm.at[pl.ds(base + j * CHUNK, CHUNK)],
                    wsems[b],
                )
                jn = j + PRIME
                bn = (b + PRIME) % NBUF

                @pl.when(jn < nchunks)
                def _():
                    @pl.when(j >= NBUF - PRIME)
                    def _():
                        pltpu.make_async_copy(
                            rows_v.at[bn],
                            out_hbm.at[pl.ds(base, CHUNK)],
                            wsems[bn],
                        ).wait()

                    pltpu.async_copy(
                        table_hbm.at[idx_v.at[jn]], rows_v.at[bn], gsems[bn]
                    )

            return carry

        lax.fori_loop(0, ngroups, group, 0)

        for b in range(NBUF):
            pltpu.make_async_copy(
                rows_v.at[b], out_hbm.at[pl.ds(base, CHUNK)], wsems[b]
            ).wait()

    return gather_kernel


def kernel(x, table):
    total = x.shape[0] * x.shape[1]
    d = table.shape[1]
    idx = x.astype(jnp.int32).reshape(NW, total // (NW * CHUNK), CHUNK)
    out = _make_sc_gather(total, d)(idx, table)
    return out.reshape(x.shape[0], x.shape[1], d)
